# baseline (speedup 1.0000x reference)
import jax
import jax.numpy as jnp
from jax import lax
from jax.experimental import pallas as pl
from jax.experimental.pallas import tpu as pltpu
from jax.experimental.pallas import tpu_sc as plsc

_IOU_T = 0.5
_SCORE_T = 0.8
_MAXDET = 300
_OUTW = 384
_B = 4
_NPAD = 5120
_L = 16
_NCH = _NPAD // _L
_NCHV = _NCH // _L
_KCAP = 304
_KV = _KCAP // _L
_OV = _OUTW // _L

_NEG = float("-inf")

_GDN = lax.GatherDimensionNumbers(offset_dims=(), collapsed_slice_dims=(0,),
                                  start_index_map=(0,))


def _sc_nms_body(scores_hbm, x1_hbm, y1_hbm, x2_hbm, y2_hbm, cls_hbm,
                 out_s, out_x1, out_y1, out_x2, out_y2, out_c, out_n,
                 sraw, x1r, y1r, x2r, y2r, clsr,
                 kx1, ky1, kx2, ky2, ka,
                 os_, ox1, oy1, ox2, oy2, oc, onum,
                 cmv, l2v, stt, mcsh, mcbuf):
    c = lax.axis_index("c")
    s = lax.axis_index("s")
    active = (c == 0) & (s < _B)
    bb = jnp.minimum(s, _B - 1)
    iot = lax.iota(jnp.int32, _L)

    def fsplat(v):
        return jnp.full((_L,), v, jnp.float32)

    def isplat(v):
        return jnp.full((_L,), v, jnp.int32)

    def bfly(x, op):
        for d in (8, 4, 2, 1):
            g = lax.gather(x, (iot ^ d).reshape(_L, 1), _GDN, (1,),
                           mode=lax.GatherScatterMode.PROMISE_IN_BOUNDS)
            x = op(x, g)
        return x

    negv = fsplat(_NEG)
    thrv = fsplat(_SCORE_T)
    zerov = fsplat(0.0)
    zeroiv = isplat(0)

    @pl.when(active)
    def _stage():
        pltpu.sync_copy(scores_hbm.at[bb], sraw)
        pltpu.sync_copy(x1_hbm.at[bb], x1r)
        pltpu.sync_copy(y1_hbm.at[bb], y1r)
        pltpu.sync_copy(x2_hbm.at[bb], x2r)
        pltpu.sync_copy(y2_hbm.at[bb], y2r)
        pltpu.sync_copy(cls_hbm.at[bb], clsr)

    def _maxbody(i, acc):
        sv = sraw[pl.ds(i * _L, _L)]
        msk = sv > thrv
        for pr in (x1r, y1r, x2r, y2r):
            acc = jnp.maximum(acc, jnp.where(msk, pr[pl.ds(i * _L, _L)], negv))
        return acc

    mymaxv = bfly(lax.fori_loop(0, _NCH, _maxbody, negv), jnp.maximum)
    mygated = jnp.where(active, mymaxv[0], jnp.float32(_NEG))
    mcbuf[0, :] = fsplat(mygated)
    pltpu.sync_copy(mcbuf.at[0], mcsh.at[s])
    plsc.subcore_barrier()
    pltpu.sync_copy(mcsh, mcbuf)
    mcacc = negv
    for i in range(_L):
        mcacc = jnp.maximum(mcacc, mcbuf[i, :])
    mc = bfly(mcacc, jnp.maximum)[0] + 1.0
    off = lax.convert_element_type(bb, jnp.float32) * mc
    offv = fsplat(off)

    @pl.when(active)
    def _init():
        def _cmbody(i, _):
            mv = bfly(sraw[pl.ds(i * _L, _L)], jnp.maximum)
            vb = (i // _L) * _L
            v = cmv[pl.ds(vb, _L)]
            cmv[pl.ds(vb, _L)] = jnp.where(iot == isplat(i - vb), mv, v)
            return 0

        lax.fori_loop(0, _NCH, _cmbody, 0)
        lv0 = negv
        lv1 = negv
        for j in range(_NCHV):
            mj = bfly(cmv[pl.ds(j * _L, _L)], jnp.maximum)
            if j < _L:
                lv0 = jnp.where(iot == isplat(j), mj, lv0)
            else:
                lv1 = jnp.where(iot == isplat(j - _L), mj, lv1)
        l2v[pl.ds(0, _L)] = lv0
        l2v[pl.ds(_L, _L)] = lv1
        for k in range(_KV):
            kx1[pl.ds(k * _L, _L)] = zerov
            ky1[pl.ds(k * _L, _L)] = zerov
            kx2[pl.ds(k * _L, _L)] = zerov
            ky2[pl.ds(k * _L, _L)] = zerov
            ka[pl.ds(k * _L, _L)] = zerov
        for k in range(_OV):
            os_[pl.ds(k * _L, _L)] = zerov
            ox1[pl.ds(k * _L, _L)] = zerov
            oy1[pl.ds(k * _L, _L)] = zerov
            ox2[pl.ds(k * _L, _L)] = zerov
            oy2[pl.ds(k * _L, _L)] = zerov
            oc[pl.ds(k * _L, _L)] = zeroiv

    stt[...] = jnp.where(iot == isplat(1),
                         isplat(jnp.where(active, jnp.int32(0), jnp.int32(1))),
                         zeroiv)
    _BLK = 8

    def selbody(i, _):
        st = stt[pl.ds(0, _L)]
        done0 = st[1] > 0

        @pl.when(jnp.logical_not(done0))
        def _block():
            cnt = st[0]
            gm = jnp.float32(0.0)
            for _k in range(_BLK):
                a0 = l2v[pl.ds(0, _L)]
                a1 = l2v[pl.ds(_L, _L)]
                gmv = bfly(jnp.maximum(a0, a1), jnp.maximum)
                gm = gmv[0]
                jc = jnp.minimum(
                    jnp.where(a0 == gmv, iot, isplat(2 * _L)),
                    jnp.where(a1 == gmv, iot + isplat(_L), isplat(2 * _L)))
                jv = bfly(jc, jnp.minimum)[0]
                cmj = cmv[pl.ds(jv * _L, _L)]
                cc = jnp.where(cmj == gmv, iot + isplat(jv * _L), isplat(_NCH))
                cidx = bfly(cc, jnp.minimum)[0]
                base = cidx * _L
                sv = sraw[pl.ds(base, _L)]
                lidxv = bfly(jnp.where(sv == gmv, iot, isplat(_L)), jnp.minimum)
                oh = iot == lidxv
                lix = lidxv.reshape(_L, 1)
                pm = lax.GatherScatterMode.PROMISE_IN_BOUNDS
                gx1v = lax.gather(x1r[pl.ds(base, _L)], lix, _GDN, (1,), mode=pm)
                gy1v = lax.gather(y1r[pl.ds(base, _L)], lix, _GDN, (1,), mode=pm)
                gx2v = lax.gather(x2r[pl.ds(base, _L)], lix, _GDN, (1,), mode=pm)
                gy2v = lax.gather(y2r[pl.ds(base, _L)], lix, _GDN, (1,), mode=pm)
                cclsv = lax.gather(clsr[pl.ds(base, _L)], lix, _GDN, (1,), mode=pm)
                cx1v = gx1v + offv
                cy1v = gy1v + offv
                cx2v = gx2v + offv
                cy2v = gy2v + offv
                cav = (jnp.maximum(cx2v - cx1v, zerov) *
                       jnp.maximum(cy2v - cy1v, zerov))
                supv = zerov
                for k in range(_KV):
                    k1 = kx1[pl.ds(k * _L, _L)]
                    l1 = ky1[pl.ds(k * _L, _L)]
                    k2 = kx2[pl.ds(k * _L, _L)]
                    l2 = ky2[pl.ds(k * _L, _L)]
                    kav = ka[pl.ds(k * _L, _L)]
                    iw = jnp.maximum(jnp.minimum(cx2v, k2) - jnp.maximum(cx1v, k1), zerov)
                    ih = jnp.maximum(jnp.minimum(cy2v, l2) - jnp.maximum(cy1v, l1), zerov)
                    inter = iw * ih
                    union = cav + kav - inter
                    iou = inter / jnp.maximum(union, fsplat(1e-9))
                    supv = jnp.maximum(supv, iou)
                sup = bfly(supv, jnp.maximum)[0] > _IOU_T
                act = (gm > _SCORE_T) & (cnt < _MAXDET)
                keep = act & jnp.logical_not(sup)
                wb = (cnt // _L) * _L
                wsl = jnp.where(keep, cnt - wb, jnp.int32(-1))
                ohw = iot == isplat(wsl)
                os_[pl.ds(wb, _L)] = jnp.where(ohw, gmv, os_[pl.ds(wb, _L)])
                ox1[pl.ds(wb, _L)] = jnp.where(ohw, gx1v, ox1[pl.ds(wb, _L)])
                oy1[pl.ds(wb, _L)] = jnp.where(ohw, gy1v, oy1[pl.ds(wb, _L)])
                ox2[pl.ds(wb, _L)] = jnp.where(ohw, gx2v, ox2[pl.ds(wb, _L)])
                oy2[pl.ds(wb, _L)] = jnp.where(ohw, gy2v, oy2[pl.ds(wb, _L)])
                oc[pl.ds(wb, _L)] = jnp.where(ohw, cclsv, oc[pl.ds(wb, _L)])
                kx1[pl.ds(wb, _L)] = jnp.where(ohw, cx1v, kx1[pl.ds(wb, _L)])
                ky1[pl.ds(wb, _L)] = jnp.where(ohw, cy1v, ky1[pl.ds(wb, _L)])
                kx2[pl.ds(wb, _L)] = jnp.where(ohw, cx2v, kx2[pl.ds(wb, _L)])
                ky2[pl.ds(wb, _L)] = jnp.where(ohw, cy2v, ky2[pl.ds(wb, _L)])
                ka[pl.ds(wb, _L)] = jnp.where(ohw, cav, ka[pl.ds(wb, _L)])
                rsl = jnp.where(act, lidxv[0], jnp.int32(-1))
                ohr = iot == isplat(rsl)
                newsv = jnp.where(ohr, negv, sv)
                sraw[pl.ds(base, _L)] = newsv
                newmv = bfly(newsv, jnp.maximum)
                cb = (cidx // _L) * _L
                csl = jnp.where(act, cidx - cb, jnp.int32(-1))
                ohc = iot == isplat(csl)
                cmv[pl.ds(cb, _L)] = jnp.where(ohc, newmv, cmv[pl.ds(cb, _L)])
                l2newv = bfly(cmv[pl.ds(cb, _L)], jnp.maximum)
                jb = cidx // _L
                lb = (jb // _L) * _L
                ohl = iot == isplat(jnp.where(act, jb - lb, jnp.int32(-1)))
                l2v[pl.ds(lb, _L)] = jnp.where(ohl, l2newv, l2v[pl.ds(lb, _L)])
                cnt = jnp.where(keep, cnt + 1, cnt)
            done2 = (gm <= _SCORE_T) | (cnt >= _MAXDET)
            d2i = jnp.where(done2, jnp.int32(1), jnp.int32(0))
            st2 = jnp.where(iot == isplat(0), isplat(cnt),
                            jnp.where(iot == isplat(1), isplat(d2i), st))
            stt[pl.ds(0, _L)] = st2

        return 0

    lax.fori_loop(0, (_NPAD + 2 + _BLK - 1) // _BLK, selbody, 0)
    onum[...] = isplat(stt[pl.ds(0, _L)][0])

    @pl.when(active)
    def _writeback():
        pltpu.sync_copy(os_, out_s.at[bb])
        pltpu.sync_copy(ox1, out_x1.at[bb])
        pltpu.sync_copy(oy1, out_y1.at[bb])
        pltpu.sync_copy(ox2, out_x2.at[bb])
        pltpu.sync_copy(oy2, out_y2.at[bb])
        pltpu.sync_copy(oc, out_c.at[bb])
        pltpu.sync_copy(onum, out_n.at[bb])


def _nms_call(scores_p, x1, y1, x2, y2, cls_p):
    mesh = plsc.VectorSubcoreMesh(core_axis_name="c", subcore_axis_name="s")
    fn = pl.kernel(
        _sc_nms_body,
        out_type=[
            jax.ShapeDtypeStruct((_B, _OUTW), jnp.float32),
            jax.ShapeDtypeStruct((_B, _OUTW), jnp.float32),
            jax.ShapeDtypeStruct((_B, _OUTW), jnp.float32),
            jax.ShapeDtypeStruct((_B, _OUTW), jnp.float32),
            jax.ShapeDtypeStruct((_B, _OUTW), jnp.float32),
            jax.ShapeDtypeStruct((_B, _OUTW), jnp.int32),
            jax.ShapeDtypeStruct((_B, _L), jnp.int32),
        ],
        mesh=mesh,
        scratch_types=[
            pltpu.VMEM((_NPAD,), jnp.float32),
            pltpu.VMEM((_NPAD,), jnp.float32),
            pltpu.VMEM((_NPAD,), jnp.float32),
            pltpu.VMEM((_NPAD,), jnp.float32),
            pltpu.VMEM((_NPAD,), jnp.float32),
            pltpu.VMEM((_NPAD,), jnp.int32),
            pltpu.VMEM((_KCAP,), jnp.float32),
            pltpu.VMEM((_KCAP,), jnp.float32),
            pltpu.VMEM((_KCAP,), jnp.float32),
            pltpu.VMEM((_KCAP,), jnp.float32),
            pltpu.VMEM((_KCAP,), jnp.float32),
            pltpu.VMEM((_OUTW,), jnp.float32),
            pltpu.VMEM((_OUTW,), jnp.float32),
            pltpu.VMEM((_OUTW,), jnp.float32),
            pltpu.VMEM((_OUTW,), jnp.float32),
            pltpu.VMEM((_OUTW,), jnp.float32),
            pltpu.VMEM((_OUTW,), jnp.int32),
            pltpu.VMEM((_L,), jnp.int32),
            pltpu.VMEM((_NCH,), jnp.float32),
            pltpu.VMEM((2 * _L,), jnp.float32),
            pltpu.VMEM((_L,), jnp.int32),
            pltpu.VMEM_SHARED((_L, _L), jnp.float32),
            pltpu.VMEM((_L, _L), jnp.float32),
        ],
    )
    return fn(scores_p, x1, y1, x2, y2, cls_p)


def kernel(scores, boxes, classes):
    B_, N_ = scores.shape
    pad = _NPAD - N_
    scores_p = jnp.pad(scores, ((0, 0), (0, pad)), constant_values=-1.0)
    x1 = jnp.pad(boxes[..., 0], ((0, 0), (0, pad)))
    y1 = jnp.pad(boxes[..., 1], ((0, 0), (0, pad)))
    x2 = jnp.pad(boxes[..., 2], ((0, 0), (0, pad)))
    y2 = jnp.pad(boxes[..., 3], ((0, 0), (0, pad)))
    cls_p = jnp.pad(classes.astype(jnp.int32), ((0, 0), (0, pad)))
    out_s, ox1, oy1, ox2, oy2, out_c, out_n = _nms_call(
        scores_p, x1, y1, x2, y2, cls_p)
    dummy = jnp.full((B_, _MAXDET), -1, jnp.int32)
    boxes_o = jnp.stack([ox1, oy1, ox2, oy2], axis=-1)[:, :_MAXDET, :]
    return (dummy,
            out_s[:, :_MAXDET],
            boxes_o,
            out_c[:, :_MAXDET],
            out_n[:B_, 0])
